# Initial kernel scaffold; baseline (speedup 1.0000x reference)
import jax
import jax.numpy as jnp
from jax.experimental import pallas as pl


def kernel(z, pos, edge_index, t, params):
    raise NotImplementedError("write your pallas kernel here")


# speedup vs baseline: 2.2285x; 2.2285x over previous
import functools

import jax
import jax.numpy as jnp
from jax import lax
from jax.experimental import pallas as pl
from jax.experimental.pallas import tpu as pltpu
from jax.experimental.pallas import tpu_sc as plsc

N = 10000
E = 160000
F = 64
HD = 256
NL = 6
WT = 80
CH = 128
NCH = E // CH
NC = 2
NS = 16
NW = NC * NS
SLAB = N // NS
TE = 1600
GE = E // TE
TN = 2000
GN = N // TN

_f32 = jnp.float32


def _silu(x):
    return x * (1.0 / (1.0 + jnp.exp(-x)))


def _layernorm(x, g, b):
    m = jnp.mean(x, axis=-1, keepdims=True)
    v = jnp.mean((x - m) ** 2, axis=-1, keepdims=True)
    return (x - m) / jnp.sqrt(v + 1e-5) * g + b


def _tvec_body(t_ref, w1_ref, b1_ref, w2_ref, b2_ref, tw_ref, tb_ref, out_ref):
    te = _silu(jnp.dot(t_ref[...], w1_ref[...], preferred_element_type=_f32)
               + b1_ref[...])
    temb = jnp.dot(te, w2_ref[...], preferred_element_type=_f32) + b2_ref[...]
    st = _silu(temb)
    for l in range(NL):
        out_ref[l:l + 1, :] = (
            jnp.dot(st, tw_ref[l], preferred_element_type=_f32)
            + tb_ref[l:l + 1, :])


def _tvec(t11, tw1, tb1, tw2, tb2, time_w, time_b):
    return pl.pallas_call(
        _tvec_body,
        out_shape=jax.ShapeDtypeStruct((NL, F), _f32),
    )(t11, tw1, tb1, tw2, tb2, time_w, time_b)


def _init_body(z_ref, pos_ref, emb_ref, tv_ref, ht_ref, p16_ref):
    zt = z_ref[0, 0, :]
    oh = (zt[:, None] == lax.broadcasted_iota(jnp.int32, (TN, 128), 1))
    h0 = jnp.dot(oh.astype(_f32), emb_ref[...], preferred_element_type=_f32)
    h0 = h0 + tv_ref[0:1, :]
    p = pos_ref[...]
    zpad = jnp.zeros((TN, WT - F - 3), _f32)
    ht_ref[...] = jnp.concatenate([h0, p, zpad], axis=1)
    p16_ref[...] = jnp.concatenate([p, zpad], axis=1)


def _init(z3d, pos, emb_pad, tvec):
    return pl.pallas_call(
        _init_body,
        grid=(GN,),
        in_specs=[
            pl.BlockSpec((1, 1, TN), lambda i: (i, 0, 0)),
            pl.BlockSpec((TN, 3), lambda i: (i, 0)),
            pl.BlockSpec((128, F), lambda i: (0, 0)),
            pl.BlockSpec((NL, F), lambda i: (0, 0)),
        ],
        out_specs=[
            pl.BlockSpec((TN, WT), lambda i: (i, 0)),
            pl.BlockSpec((TN, 16), lambda i: (i, 0)),
        ],
        out_shape=[
            jax.ShapeDtypeStruct((N, WT), _f32),
            jax.ShapeDtypeStruct((N, 16), _f32),
        ],
    )(z3d, pos, emb_pad, tvec)


def _sc_gather(ht, row2d, col2d):
    mesh = plsc.VectorSubcoreMesh(core_axis_name="c", subcore_axis_name="s")

    @functools.partial(
        pl.kernel,
        out_type=[
            jax.ShapeDtypeStruct((E, WT), _f32),
            jax.ShapeDtypeStruct((E, WT), _f32),
        ],
        mesh=mesh,
        compiler_params=pltpu.CompilerParams(use_tc_tiling_on_sc=False),
        scratch_types=[
            pltpu.VMEM((CH,), jnp.int32),
            pltpu.VMEM((CH, WT), _f32),
            pltpu.SemaphoreType.DMA,
        ],
    )
    def k(ht_h, row_h, col_h, orow_h, ocol_h, idx_v, buf_v, sem):
        w = lax.axis_index("s") * NC + lax.axis_index("c")

        def body(i, carry):
            ch = w + i * NW

            @pl.when(ch < NCH)
            def _():
                pltpu.sync_copy(row_h.at[ch], idx_v)
                pltpu.async_copy(ht_h.at[idx_v], buf_v, sem).wait()
                pltpu.sync_copy(buf_v, orow_h.at[pl.ds(ch * CH, CH)])
                pltpu.sync_copy(col_h.at[ch], idx_v)
                pltpu.async_copy(ht_h.at[idx_v], buf_v, sem).wait()
                pltpu.sync_copy(buf_v, ocol_h.at[pl.ds(ch * CH, CH)])

            return carry

        lax.fori_loop(0, (NCH + NW - 1) // NW, body, 0)

    return k(ht, row2d, col2d)


def _edge_body(hr_ref, hc_ref, w1r_ref, w1c_ref, w1d_ref, b1_ref, g_ref,
               bb_ref, w2_ref, b2_ref, cw1_ref, cb1_ref, cw2_ref,
               msg_ref, tr_ref):
    hr = hr_ref[:, :F]
    hc = hc_ref[:, :F]
    pr = hr_ref[:, F:F + 3]
    pc = hc_ref[:, F:F + 3]
    rel = pr - pc
    dsq = jnp.sum(rel * rel, axis=1, keepdims=True) + 1e-12
    x = (jnp.dot(hr, w1r_ref[...], preferred_element_type=_f32)
         + jnp.dot(hc, w1c_ref[...], preferred_element_type=_f32)
         + dsq * w1d_ref[...] + b1_ref[...])
    x = _layernorm(x, g_ref[...], bb_ref[...])
    x = _silu(x)
    x = jnp.dot(x, w2_ref[...], preferred_element_type=_f32) + b2_ref[...]
    msg = _silu(x)
    dist = jnp.sqrt(dsq)
    msg = msg * jnp.exp(dist * (-1.0 / 5.0))
    cw = _silu(jnp.dot(msg, cw1_ref[...], preferred_element_type=_f32)
               + cb1_ref[...])
    cwf = jnp.tanh(jnp.sum(cw * cw2_ref[...], axis=1, keepdims=True))
    rpn = rel / (dist + 1e-6)
    tr = rpn * cwf
    msg_ref[0] = msg[:, :128]
    msg_ref[1] = msg[:, 128:]
    tr_ref[...] = jnp.concatenate([tr, jnp.zeros((TE, 13), _f32)], axis=1)


def _edge(hrowp, hcolp, wts):
    full = lambda shape: pl.BlockSpec(shape, lambda i: tuple(0 for _ in shape))
    return pl.pallas_call(
        _edge_body,
        grid=(GE,),
        in_specs=[
            pl.BlockSpec((TE, WT), lambda i: (i, 0)),
            pl.BlockSpec((TE, WT), lambda i: (i, 0)),
            full((F, HD)), full((F, HD)), full((1, HD)), full((1, HD)),
            full((1, HD)), full((1, HD)), full((HD, HD)), full((1, HD)),
            full((HD, HD)), full((1, HD)), full((1, HD)),
        ],
        out_specs=[
            pl.BlockSpec((2, TE, 128), lambda i: (0, i, 0)),
            pl.BlockSpec((TE, 16), lambda i: (i, 0)),
        ],
        out_shape=[
            jax.ShapeDtypeStruct((2, E, 128), _f32),
            jax.ShapeDtypeStruct((E, 16), _f32),
        ],
    )(hrowp, hcolp, *wts)


def _sc_scatter(msgS, tr16, row2d, p16_in):
    mesh = plsc.VectorSubcoreMesh(core_axis_name="c", subcore_axis_name="s")

    @functools.partial(
        pl.kernel,
        out_type=[
            jax.ShapeDtypeStruct((NC, N, 128), _f32),
            jax.ShapeDtypeStruct((N, 16), _f32),
        ],
        mesh=mesh,
        compiler_params=pltpu.CompilerParams(use_tc_tiling_on_sc=False),
        scratch_types=[
            pltpu.VMEM((CH,), jnp.int32),
            pltpu.VMEM((CH, 128), _f32),
            pltpu.VMEM((CH, 16), _f32),
            pltpu.VMEM((25, 128), _f32),
            pltpu.VMEM_SHARED((N, 128), _f32),
            pltpu.VMEM_SHARED((N, 16), _f32),
        ],
    )
    def k(msg_h, tr_h, row_h, p16_h, agg_h, p16o_h,
          idx_v, mbuf, tbuf, zbuf, accA, accP):
        c = lax.axis_index("c")
        s = lax.axis_index("s")
        zv = jnp.zeros((16,), _f32)
        for j in range(25):
            for kk in range(8):
                zbuf[j, pl.ds(kk * 16, 16)] = zv

        def zero_body(q, carry):
            pltpu.sync_copy(zbuf, accA.at[pl.ds(s * SLAB + q * 25, 25)])
            return carry

        lax.fori_loop(0, SLAB // 25, zero_body, 0)

        @pl.when(c == 0)
        def _():
            pltpu.sync_copy(p16_h.at[pl.ds(s * SLAB, SLAB)],
                            accP.at[pl.ds(s * SLAB, SLAB)])

        plsc.subcore_barrier()

        def body(i, carry):
            ch = s + i * NS

            @pl.when(ch < NCH)
            def _():
                pltpu.sync_copy(row_h.at[ch], idx_v)
                pltpu.sync_copy(msg_h.at[c, pl.ds(ch * CH, CH)], mbuf)
                pltpu.sync_copy(mbuf, accA.at[idx_v], add=True)

                @pl.when(c == 0)
                def _():
                    pltpu.sync_copy(tr_h.at[pl.ds(ch * CH, CH)], tbuf)
                    pltpu.sync_copy(tbuf, accP.at[idx_v], add=True)

            return carry

        lax.fori_loop(0, (NCH + NS - 1) // NS, body, 0)
        plsc.subcore_barrier()
        pltpu.sync_copy(accA.at[pl.ds(s * SLAB, SLAB)],
                        agg_h.at[c, pl.ds(s * SLAB, SLAB)])

        @pl.when(c == 0)
        def _():
            pltpu.sync_copy(accP.at[pl.ds(s * SLAB, SLAB)],
                            p16o_h.at[pl.ds(s * SLAB, SLAB)])

    return k(msgS, tr16, row2d, p16_in)


def _node_body(ht_ref, agg_ref, p16_ref, tv_ref, wa_ref, wb_ref, wc_ref,
               b1_ref, w2_ref, b2_ref, g_ref, bb_ref, out_ref):
    h = ht_ref[:, :F]
    hu = _silu(jnp.dot(h, wa_ref[...], preferred_element_type=_f32)
               + jnp.dot(agg_ref[0], wb_ref[...], preferred_element_type=_f32)
               + jnp.dot(agg_ref[1], wc_ref[...], preferred_element_type=_f32)
               + b1_ref[...])
    hu = jnp.dot(hu, w2_ref[...], preferred_element_type=_f32) + b2_ref[...]
    hn = _layernorm(h + hu, g_ref[...], bb_ref[...]) + tv_ref[...]
    out_ref[...] = jnp.concatenate(
        [hn, p16_ref[:, :3], jnp.zeros((TN, WT - F - 3), _f32)], axis=1)


def _node(ht, aggS, p16, tv_next, wts):
    full = lambda shape: pl.BlockSpec(shape, lambda i: tuple(0 for _ in shape))
    return pl.pallas_call(
        _node_body,
        grid=(GN,),
        in_specs=[
            pl.BlockSpec((TN, WT), lambda i: (i, 0)),
            pl.BlockSpec((2, TN, 128), lambda i: (0, i, 0)),
            pl.BlockSpec((TN, 16), lambda i: (i, 0)),
            full((1, F)),
            full((F, HD)), full((128, HD)), full((128, HD)), full((1, HD)),
            full((HD, F)), full((1, F)), full((1, F)), full((1, F)),
        ],
        out_specs=pl.BlockSpec((TN, WT), lambda i: (i, 0)),
        out_shape=jax.ShapeDtypeStruct((N, WT), _f32),
    )(ht, aggS, p16, tv_next, *wts)


def _heads_body(ht_ref, p16_ref, pos0_ref,
                hw1_ref, hb1_ref, hw2_ref, hb2_ref,
                ew1_ref, eb1_ref, ew2_ref, eb2_ref,
                sw1_ref, sb1_ref, sw2_ref, sb2_ref,
                noise_ref, misc_ref):
    h = ht_ref[:, :F]
    noise_ref[...] = p16_ref[:, :3] - pos0_ref[...]
    hg = jnp.mean(h, axis=0, keepdims=True)

    def head(w1, b1, w2v, b2):
        y = _silu(jnp.dot(hg, w1, preferred_element_type=_f32) + b1)
        return jnp.sum(y * w2v, axis=1, keepdims=True) + b2

    her = head(hw1_ref[...], hb1_ref[...], hw2_ref[...], hb2_ref[...])
    en = head(ew1_ref[...], eb1_ref[...], ew2_ref[...], eb2_ref[...])
    sy = head(sw1_ref[...], sb1_ref[...], sw2_ref[...], sb2_ref[...])
    sy = 1.0 / (1.0 + jnp.exp(-sy))
    misc_ref[...] = jnp.concatenate([
        jnp.broadcast_to(her, (1, 128)),
        jnp.broadcast_to(en, (1, 128)),
        jnp.broadcast_to(sy, (1, 128)),
        jnp.zeros((5, 128), _f32),
    ], axis=0)


def _heads(ht, p16, pos0, head_wts):
    return pl.pallas_call(
        _heads_body,
        out_shape=[
            jax.ShapeDtypeStruct((N, 3), _f32),
            jax.ShapeDtypeStruct((8, 128), _f32),
        ],
    )(ht, p16, pos0, *head_wts)


def kernel(z, pos, edge_index, t, params):
    p = params
    row2d = edge_index[0].reshape(NCH, CH)
    col2d = edge_index[1].reshape(NCH, CH)

    time_w = jnp.stack([l['time_W'] for l in p['layers']])
    time_b = jnp.stack([l['time_b'] for l in p['layers']])
    tvec = _tvec(t.reshape(1, 1),
                 p['t_W1'], p['t_b1'].reshape(1, -1),
                 p['t_W2'], p['t_b2'].reshape(1, -1),
                 time_w, time_b)

    emb_pad = jnp.zeros((128, F), _f32).at[:119].set(p['emb'])
    z3d = z.astype(jnp.int32).reshape(GN, 1, TN)
    ht, p16 = _init(z3d, pos, emb_pad, tvec)

    zrow = jnp.zeros((1, F), _f32)
    for li, l in enumerate(p['layers']):
        hrowp, hcolp = _sc_gather(ht, row2d, col2d)
        edge_wts = (
            l['e_W1'][:F], l['e_W1'][F:2 * F], l['e_W1'][2 * F:2 * F + 1],
            l['e_b1'].reshape(1, -1), l['e_ln_g'].reshape(1, -1),
            l['e_ln_b'].reshape(1, -1), l['e_W2'], l['e_b2'].reshape(1, -1),
            l['c_W1'], l['c_b1'].reshape(1, -1), l['c_W2'].reshape(1, -1),
        )
        msgS, tr16 = _edge(hrowp, hcolp, edge_wts)
        aggS, p16 = _sc_scatter(msgS, tr16, row2d, p16)
        tv_next = tvec[li + 1:li + 2] if li + 1 < NL else zrow
        node_wts = (
            l['n_W1'][:F], l['n_W1'][F:F + 128], l['n_W1'][F + 128:],
            l['n_b1'].reshape(1, -1), l['n_W2'], l['n_b2'].reshape(1, -1),
            l['ln_g'].reshape(1, -1), l['ln_b'].reshape(1, -1),
        )
        ht = _node(ht, aggS, p16, tv_next, node_wts)

    head_wts = []
    for nm in ['her', 'energy', 'synth']:
        head_wts += [p[nm + '_W1'], p[nm + '_b1'].reshape(1, -1),
                     p[nm + '_W2'].reshape(1, -1), p[nm + '_b2'].reshape(1, 1)]
    noise, misc = _heads(ht, p16, pos, head_wts)
    return (noise, misc[0:1, 0:1], misc[1:2, 0:1], misc[2:3, 0:1])
